# baseline (speedup 1.0000x reference)
import functools

import jax
import jax.numpy as jnp
from jax import lax
from jax.experimental import pallas as pl
from jax.experimental.pallas import tpu as pltpu
from jax.experimental.pallas import tpu_sc as plsc

B = 1024
W = 20
E = 64
V = 100000

_NC = 2
_NS = 16
_NW = _NC * _NS
_BPW = B // _NW
_IPW = _BPW * W
_CHUNK = 128
_NCHUNK = _IPW // _CHUNK
_LANES = E // 16

BLOCK_V = 2048
_NBLK = (V + BLOCK_V - 1) // BLOCK_V


def _gather_mean_body(idx_hbm, table_hbm, h_hbm, idx_v, rows_v, hsum_v, sem):
    wid = lax.axis_index("s") * _NC + lax.axis_index("c")
    pltpu.sync_copy(idx_hbm.at[wid], idx_v)
    copies = [
        pltpu.async_copy(
            table_hbm.at[idx_v.at[j]],
            rows_v.at[pl.ds(j * _CHUNK, _CHUNK)],
            sem,
        )
        for j in range(_NCHUNK)
    ]
    for c in copies:
        c.wait()

    inv_w = jnp.float32(1.0 / W)

    def row_body(b, carry):
        def w_body(w, accs):
            r = b * W + w
            return tuple(
                accs[c] + rows_v[r, pl.ds(c * 16, 16)] for c in range(_LANES)
            )

        accs = lax.fori_loop(
            0, W, w_body, tuple(jnp.zeros((16,), jnp.float32) for _ in range(_LANES))
        )
        for c in range(_LANES):
            hsum_v[b, pl.ds(c * 16, 16)] = accs[c] * inv_w
        return carry

    lax.fori_loop(0, _BPW, row_body, 0)
    pltpu.sync_copy(hsum_v, h_hbm.at[pl.ds(wid * _BPW, _BPW)])


@functools.lru_cache(maxsize=1)
def _gather_mean():
    return pl.kernel(
        _gather_mean_body,
        out_type=jax.ShapeDtypeStruct((B, E), jnp.float32),
        mesh=plsc.VectorSubcoreMesh(core_axis_name="c", subcore_axis_name="s"),
        scratch_types=[
            pltpu.VMEM((_NCHUNK, _CHUNK), jnp.int32),
            pltpu.VMEM((_IPW, E), jnp.float32),
            pltpu.VMEM((_BPW, E), jnp.float32),
            pltpu.SemaphoreType.DMA,
        ],
        compiler_params=pltpu.CompilerParams(use_tc_tiling_on_sc=False),
    )


def _mm_body(h_ref, w_ref, b_ref, o_ref):
    o_ref[...] = (
        lax.dot_general(
            h_ref[...],
            w_ref[...],
            (((1,), (1,)), ((), ())),
            preferred_element_type=jnp.float32,
        )
        + b_ref[...]
    )


def kernel(inputs, embedding, linear_w, linear_b):
    idx = inputs.astype(jnp.int32).reshape(_NW, _NCHUNK, _CHUNK)
    h = _gather_mean()(idx, embedding)
    out = pl.pallas_call(
        _mm_body,
        grid=(_NBLK,),
        in_specs=[
            pl.BlockSpec((B, E), lambda i: (0, 0)),
            pl.BlockSpec((BLOCK_V, E), lambda i: (i, 0)),
            pl.BlockSpec((1, BLOCK_V), lambda i: (0, i)),
        ],
        out_specs=pl.BlockSpec((B, BLOCK_V), lambda i: (0, i)),
        out_shape=jax.ShapeDtypeStruct((B, V), jnp.float32),
    )(h, linear_w, linear_b.reshape(1, V))
    return out
